# Initial kernel scaffold; baseline (speedup 1.0000x reference)
import functools

import jax
import jax.numpy as jnp
from jax import lax
from jax.experimental import pallas as pl
from jax.experimental.pallas import tpu as pltpu
from jax.experimental.pallas import tpu_sc as plsc

_N = 10000
_E = 320000
_H = 128
_NC = 2
_NS = 16
_NW = _NC * _NS
_EPW = _E // _NW
_CHUNK = 80
_NCHUNK = _EPW // _CHUNK
_RPT = _N // _NS
_BN = 1000
_GRID = _N // _BN

_f32 = jnp.float32


def _sc_mesh():
    return plsc.VectorSubcoreMesh(core_axis_name="c", subcore_axis_name="s")


@functools.partial(
    pl.kernel,
    out_type=(
        jax.ShapeDtypeStruct((_NC, _N, _H), _f32),
        jax.ShapeDtypeStruct((_NC, _N, 16), _f32),
    ),
    mesh=_sc_mesh(),
    scratch_types=[
        pltpu.VMEM((_NCHUNK, _CHUNK), jnp.int32),
        pltpu.VMEM((_NCHUNK, _CHUNK), jnp.int32),
        pltpu.VMEM((_CHUNK, _H), _f32),
        pltpu.VMEM((_CHUNK, 16), _f32),
        pltpu.VMEM_SHARED((_N, _H), _f32),
        pltpu.VMEM_SHARED((_N, 16), _f32),
        pltpu.SemaphoreType.DMA,
    ],
)
def _sc_scatter_cnt(y_hbm, ei_hbm, zh_hbm, z16_hbm, ones_hbm,
                    agg_out, cnt_out, sidx, didx, rows, ones, acc, cacc, sem):
    c = lax.axis_index("c")
    s = lax.axis_index("s")
    rbase = s * _RPT
    pltpu.sync_copy(zh_hbm.at[pl.ds(rbase, _RPT)], acc.at[pl.ds(rbase, _RPT)])
    pltpu.sync_copy(z16_hbm.at[pl.ds(rbase, _RPT)], cacc.at[pl.ds(rbase, _RPT)])
    tid = s * _NC + c
    pltpu.sync_copy(ei_hbm.at[0, pl.ds(tid * _NCHUNK, _NCHUNK)], sidx)
    pltpu.sync_copy(ei_hbm.at[1, pl.ds(tid * _NCHUNK, _NCHUNK)], didx)
    pltpu.sync_copy(ones_hbm, ones)
    plsc.subcore_barrier()

    def body(i, carry):
        pltpu.async_copy(y_hbm.at[sidx.at[i]], rows, sem).wait()
        pltpu.sync_copy(rows, acc.at[didx.at[i]], add=True)
        pltpu.sync_copy(ones, cacc.at[didx.at[i]], add=True)
        return carry

    lax.fori_loop(0, _NCHUNK, body, 0)
    plsc.subcore_barrier()
    pltpu.sync_copy(acc.at[pl.ds(rbase, _RPT)],
                    agg_out.at[c, pl.ds(rbase, _RPT)])
    pltpu.sync_copy(cacc.at[pl.ds(rbase, _RPT)],
                    cnt_out.at[c, pl.ds(rbase, _RPT)])


@functools.partial(
    pl.kernel,
    out_type=jax.ShapeDtypeStruct((_NC, _N, _H), _f32),
    mesh=_sc_mesh(),
    scratch_types=[
        pltpu.VMEM((_NCHUNK, _CHUNK), jnp.int32),
        pltpu.VMEM((_NCHUNK, _CHUNK), jnp.int32),
        pltpu.VMEM((_CHUNK, _H), _f32),
        pltpu.VMEM_SHARED((_N, _H), _f32),
        pltpu.SemaphoreType.DMA,
    ],
)
def _sc_scatter(y_hbm, ei_hbm, zh_hbm, agg_out, sidx, didx, rows, acc, sem):
    c = lax.axis_index("c")
    s = lax.axis_index("s")
    rbase = s * _RPT
    pltpu.sync_copy(zh_hbm.at[pl.ds(rbase, _RPT)], acc.at[pl.ds(rbase, _RPT)])
    tid = s * _NC + c
    pltpu.sync_copy(ei_hbm.at[0, pl.ds(tid * _NCHUNK, _NCHUNK)], sidx)
    pltpu.sync_copy(ei_hbm.at[1, pl.ds(tid * _NCHUNK, _NCHUNK)], didx)
    plsc.subcore_barrier()

    def body(i, carry):
        pltpu.async_copy(y_hbm.at[sidx.at[i]], rows, sem).wait()
        pltpu.sync_copy(rows, acc.at[didx.at[i]], add=True)
        return carry

    lax.fori_loop(0, _NCHUNK, body, 0)
    plsc.subcore_barrier()
    pltpu.sync_copy(acc.at[pl.ds(rbase, _RPT)],
                    agg_out.at[c, pl.ds(rbase, _RPT)])


def _tc_pre(x, Wl, Wr, bl):
    def body(x_ref, wl_ref, wr_ref, bl_ref, y_ref, z_ref):
        h = x_ref[...]
        y_ref[...] = jnp.dot(h, wl_ref[...], preferred_element_type=_f32)
        z_ref[...] = jnp.dot(h, wr_ref[...], preferred_element_type=_f32) + bl_ref[...]

    return pl.pallas_call(
        body,
        grid=(_GRID,),
        in_specs=[
            pl.BlockSpec((_BN, _H), lambda i: (i, 0)),
            pl.BlockSpec((_H, _H), lambda i: (0, 0)),
            pl.BlockSpec((_H, _H), lambda i: (0, 0)),
            pl.BlockSpec((1, _H), lambda i: (0, 0)),
        ],
        out_specs=[pl.BlockSpec((_BN, _H), lambda i: (i, 0))] * 2,
        out_shape=[jax.ShapeDtypeStruct((_N, _H), _f32)] * 2,
    )(x, Wl, Wr, bl.reshape(1, _H))


def _combine(aggp_ref, cntp_ref, z_ref):
    agg = aggp_ref[0] + aggp_ref[1]
    cnt = cntp_ref[0] + cntp_ref[1]
    inv = 1.0 / jnp.maximum(cnt[:, 0:1], 1.0)
    out = agg * inv + z_ref[...]
    nrm = jnp.sqrt(jnp.sum(out * out, axis=1, keepdims=True))
    return out / jnp.maximum(nrm, 1e-12)


def _tc_post(aggp, cntp, z, Wl, Wr, bl):
    def body(aggp_ref, cntp_ref, z_ref, wl_ref, wr_ref, bl_ref, y_ref, z2_ref):
        h = jnp.maximum(_combine(aggp_ref, cntp_ref, z_ref), 0.0)
        y_ref[...] = jnp.dot(h, wl_ref[...], preferred_element_type=_f32)
        z2_ref[...] = jnp.dot(h, wr_ref[...], preferred_element_type=_f32) + bl_ref[...]

    return pl.pallas_call(
        body,
        grid=(_GRID,),
        in_specs=[
            pl.BlockSpec((_NC, _BN, _H), lambda i: (0, i, 0)),
            pl.BlockSpec((_NC, _BN, 16), lambda i: (0, i, 0)),
            pl.BlockSpec((_BN, _H), lambda i: (i, 0)),
            pl.BlockSpec((_H, _H), lambda i: (0, 0)),
            pl.BlockSpec((_H, _H), lambda i: (0, 0)),
            pl.BlockSpec((1, _H), lambda i: (0, 0)),
        ],
        out_specs=[pl.BlockSpec((_BN, _H), lambda i: (i, 0))] * 2,
        out_shape=[jax.ShapeDtypeStruct((_N, _H), _f32)] * 2,
    )(aggp, cntp, z, Wl, Wr, bl.reshape(1, _H))


def _tc_final(aggp, cntp, z):
    def body(aggp_ref, cntp_ref, z_ref, h_ref):
        h_ref[...] = _combine(aggp_ref, cntp_ref, z_ref)

    return pl.pallas_call(
        body,
        grid=(_GRID,),
        in_specs=[
            pl.BlockSpec((_NC, _BN, _H), lambda i: (0, i, 0)),
            pl.BlockSpec((_NC, _BN, 16), lambda i: (0, i, 0)),
            pl.BlockSpec((_BN, _H), lambda i: (i, 0)),
        ],
        out_specs=pl.BlockSpec((_BN, _H), lambda i: (i, 0)),
        out_shape=jax.ShapeDtypeStruct((_N, _H), _f32),
    )(aggp, cntp, z)


def kernel(x, edge_index, Wl0, bl0, Wr0, Wl1, bl1, Wr1, Wl2, bl2, Wr2):
    ei3 = edge_index.reshape(2, _E // _CHUNK, _CHUNK)
    zh = jnp.zeros((_N, _H), _f32)
    z16 = jnp.zeros((_N, 16), _f32)
    ones = jnp.ones((_CHUNK, 16), _f32)

    y0, z0 = _tc_pre(x, Wl0, Wr0, bl0)
    aggp0, cntp = _sc_scatter_cnt(y0, ei3, zh, z16, ones)
    y1, z1 = _tc_post(aggp0, cntp, z0, Wl1, Wr1, bl1)
    aggp1 = _sc_scatter(y1, ei3, zh)
    y2, z2 = _tc_post(aggp1, cntp, z1, Wl2, Wr2, bl2)
    aggp2 = _sc_scatter(y2, ei3, zh)
    return _tc_final(aggp2, cntp, z2)


# speedup vs baseline: 7.8814x; 7.8814x over previous
import functools

import jax
import jax.numpy as jnp
from jax import lax
from jax.experimental import pallas as pl
from jax.experimental.pallas import tpu as pltpu
from jax.experimental.pallas import tpu_sc as plsc

_N = 10000
_E = 320000
_H = 128
_NC = 2
_NS = 16
_NW = _NC * _NS
_EPW = _E // _NW
_CHUNK = 125
_NCHUNK = _EPW // _CHUNK
_NP = 10240
_RPT = _NP // _NS
_BN = 1000
_GRID = _N // _BN

_f32 = jnp.float32


def _sc_mesh():
    return plsc.VectorSubcoreMesh(core_axis_name="c", subcore_axis_name="s")


@functools.partial(
    pl.kernel,
    out_type=jax.ShapeDtypeStruct((_NC, _NP, _H), _f32),
    mesh=_sc_mesh(),
    scratch_types=[
        pltpu.VMEM((_NCHUNK, _CHUNK), jnp.int32),
        pltpu.VMEM((_CHUNK, _H), _f32),
        pltpu.VMEM_SHARED((_NP, _H), _f32),
    ],
)
def _sc_cnt(ei_hbm, zh_hbm, ones_hbm, cnt_out, didx, ones, cacc):
    c = lax.axis_index("c")
    s = lax.axis_index("s")
    rbase = pl.multiple_of(s * _RPT, 8)
    pltpu.sync_copy(zh_hbm.at[pl.ds(rbase, _RPT)], cacc.at[pl.ds(rbase, _RPT)])
    tid = s * _NC + c
    pltpu.sync_copy(ei_hbm.at[1, pl.ds(pl.multiple_of(tid * _NCHUNK, 8), _NCHUNK)], didx)
    pltpu.sync_copy(ones_hbm, ones)
    plsc.subcore_barrier()

    def body(i, carry):
        pltpu.sync_copy(ones, cacc.at[didx.at[i]], add=True)
        return carry

    lax.fori_loop(0, _NCHUNK, body, 0)
    plsc.subcore_barrier()
    pltpu.sync_copy(cacc.at[pl.ds(rbase, _RPT)],
                    cnt_out.at[c, pl.ds(rbase, _RPT)])


@functools.partial(
    pl.kernel,
    out_type=jax.ShapeDtypeStruct((_NC, _NP, _H), _f32),
    mesh=_sc_mesh(),
    scratch_types=[
        pltpu.VMEM((_NCHUNK, _CHUNK), jnp.int32),
        pltpu.VMEM((_NCHUNK, _CHUNK), jnp.int32),
        pltpu.VMEM((_CHUNK, _H), _f32),
        pltpu.VMEM_SHARED((_NP, _H), _f32),
        pltpu.SemaphoreType.DMA,
    ],
)
def _sc_scatter(y_hbm, ei_hbm, zh_hbm, agg_out, sidx, didx, rows, acc, sem):
    c = lax.axis_index("c")
    s = lax.axis_index("s")
    rbase = pl.multiple_of(s * _RPT, 8)
    pltpu.sync_copy(zh_hbm.at[pl.ds(rbase, _RPT)], acc.at[pl.ds(rbase, _RPT)])
    tid = s * _NC + c
    pltpu.sync_copy(ei_hbm.at[0, pl.ds(pl.multiple_of(tid * _NCHUNK, 8), _NCHUNK)], sidx)
    pltpu.sync_copy(ei_hbm.at[1, pl.ds(pl.multiple_of(tid * _NCHUNK, 8), _NCHUNK)], didx)
    plsc.subcore_barrier()

    def body(i, carry):
        pltpu.async_copy(y_hbm.at[sidx.at[i]], rows, sem).wait()
        pltpu.sync_copy(rows, acc.at[didx.at[i]], add=True)
        return carry

    lax.fori_loop(0, _NCHUNK, body, 0)
    plsc.subcore_barrier()
    pltpu.sync_copy(acc.at[pl.ds(rbase, _RPT)],
                    agg_out.at[c, pl.ds(rbase, _RPT)])


def _tc_pre(x, Wl, Wr, bl):
    def body(x_ref, wl_ref, wr_ref, bl_ref, y_ref, z_ref):
        h = x_ref[...]
        y_ref[...] = jnp.dot(h, wl_ref[...], preferred_element_type=_f32)
        z_ref[...] = jnp.dot(h, wr_ref[...], preferred_element_type=_f32) + bl_ref[...]

    return pl.pallas_call(
        body,
        grid=(_GRID,),
        in_specs=[
            pl.BlockSpec((_BN, _H), lambda i: (i, 0)),
            pl.BlockSpec((_H, _H), lambda i: (0, 0)),
            pl.BlockSpec((_H, _H), lambda i: (0, 0)),
            pl.BlockSpec((1, _H), lambda i: (0, 0)),
        ],
        out_specs=[pl.BlockSpec((_BN, _H), lambda i: (i, 0))] * 2,
        out_shape=[jax.ShapeDtypeStruct((_N, _H), _f32)] * 2,
    )(x, Wl, Wr, bl.reshape(1, _H))


def _combine(aggp_ref, cntp_ref, z_ref):
    agg = aggp_ref[0] + aggp_ref[1]
    cnt = cntp_ref[0] + cntp_ref[1]
    inv = 1.0 / jnp.maximum(cnt, 1.0)
    out = agg * inv + z_ref[...]
    nrm = jnp.sqrt(jnp.sum(out * out, axis=1, keepdims=True))
    return out / jnp.maximum(nrm, 1e-12)


def _tc_post(aggp, cntp, z, Wl, Wr, bl):
    def body(aggp_ref, cntp_ref, z_ref, wl_ref, wr_ref, bl_ref, y_ref, z2_ref):
        h = jnp.maximum(_combine(aggp_ref, cntp_ref, z_ref), 0.0)
        y_ref[...] = jnp.dot(h, wl_ref[...], preferred_element_type=_f32)
        z2_ref[...] = jnp.dot(h, wr_ref[...], preferred_element_type=_f32) + bl_ref[...]

    return pl.pallas_call(
        body,
        grid=(_GRID,),
        in_specs=[
            pl.BlockSpec((_NC, _BN, _H), lambda i: (0, i, 0)),
            pl.BlockSpec((_NC, _BN, _H), lambda i: (0, i, 0)),
            pl.BlockSpec((_BN, _H), lambda i: (i, 0)),
            pl.BlockSpec((_H, _H), lambda i: (0, 0)),
            pl.BlockSpec((_H, _H), lambda i: (0, 0)),
            pl.BlockSpec((1, _H), lambda i: (0, 0)),
        ],
        out_specs=[pl.BlockSpec((_BN, _H), lambda i: (i, 0))] * 2,
        out_shape=[jax.ShapeDtypeStruct((_N, _H), _f32)] * 2,
    )(aggp, cntp, z, Wl, Wr, bl.reshape(1, _H))


def _tc_final(aggp, cntp, z):
    def body(aggp_ref, cntp_ref, z_ref, h_ref):
        h_ref[...] = _combine(aggp_ref, cntp_ref, z_ref)

    return pl.pallas_call(
        body,
        grid=(_GRID,),
        in_specs=[
            pl.BlockSpec((_NC, _BN, _H), lambda i: (0, i, 0)),
            pl.BlockSpec((_NC, _BN, _H), lambda i: (0, i, 0)),
            pl.BlockSpec((_BN, _H), lambda i: (i, 0)),
        ],
        out_specs=pl.BlockSpec((_BN, _H), lambda i: (i, 0)),
        out_shape=jax.ShapeDtypeStruct((_N, _H), _f32),
    )(aggp, cntp, z)


def kernel(x, edge_index, Wl0, bl0, Wr0, Wl1, bl1, Wr1, Wl2, bl2, Wr2):
    ei3 = edge_index.reshape(2, _E // _CHUNK, _CHUNK)
    zh = jnp.zeros((_NP, _H), _f32)
    ones = jnp.ones((_CHUNK, _H), _f32)

    cntp = _sc_cnt(ei3, zh, ones)
    y0, z0 = _tc_pre(x, Wl0, Wr0, bl0)
    aggp0 = _sc_scatter(y0, ei3, zh)
    y1, z1 = _tc_post(aggp0, cntp, z0, Wl1, Wr1, bl1)
    aggp1 = _sc_scatter(y1, ei3, zh)
    y2, z2 = _tc_post(aggp1, cntp, z1, Wl2, Wr2, bl2)
    aggp2 = _sc_scatter(y2, ei3, zh)
    return _tc_final(aggp2, cntp, z2)


# speedup vs baseline: 11.1141x; 1.4102x over previous
import functools

import jax
import jax.numpy as jnp
from jax import lax
from jax.experimental import pallas as pl
from jax.experimental.pallas import tpu as pltpu
from jax.experimental.pallas import tpu_sc as plsc

_N = 10000
_E = 320000
_H = 128
_NC = 2
_NS = 16
_NW = _NC * _NS
_EPW = _E // _NW
_CHUNK = 125
_NCHUNK = _EPW // _CHUNK
_QC = 16
_NQ = _NCHUNK // _QC
_RS = 3 * _QC
_NP = 10240
_RPT = _NP // _NS
_BN = 1000
_GRID = _N // _BN

_f32 = jnp.float32


def _sc_mesh():
    return plsc.VectorSubcoreMesh(core_axis_name="c", subcore_axis_name="s")


@functools.partial(
    pl.kernel,
    out_type=jax.ShapeDtypeStruct((_NC, _NP, _H), _f32),
    mesh=_sc_mesh(),
    scratch_types=[
        pltpu.VMEM((_NCHUNK, _CHUNK), jnp.int32),
        pltpu.VMEM((_CHUNK, _H), _f32),
        pltpu.VMEM_SHARED((_NP, _H), _f32),
    ],
)
def _sc_cnt(ei_hbm, zh_hbm, ones_hbm, cnt_out, didx, ones, cacc):
    c = lax.axis_index("c")
    s = lax.axis_index("s")
    rbase = pl.multiple_of(s * _RPT, 8)
    pltpu.sync_copy(zh_hbm.at[pl.ds(rbase, _RPT)], cacc.at[pl.ds(rbase, _RPT)])
    tid = s * _NC + c
    pltpu.sync_copy(ei_hbm.at[1, pl.ds(pl.multiple_of(tid * _NCHUNK, 8), _NCHUNK)], didx)
    pltpu.sync_copy(ones_hbm, ones)
    plsc.subcore_barrier()

    def body(i, carry):
        pltpu.sync_copy(ones, cacc.at[didx.at[i]], add=True)
        return carry

    lax.fori_loop(0, _NCHUNK, body, 0)
    plsc.subcore_barrier()
    pltpu.sync_copy(cacc.at[pl.ds(rbase, _RPT)],
                    cnt_out.at[c, pl.ds(rbase, _RPT)])


@functools.partial(
    pl.kernel,
    out_type=jax.ShapeDtypeStruct((_NC, _NP, _H), _f32),
    mesh=_sc_mesh(),
    scratch_types=[
        pltpu.VMEM((_RS, _CHUNK), jnp.int32),
        pltpu.VMEM((_RS, _CHUNK), jnp.int32),
        pltpu.VMEM((_CHUNK, _H), _f32),
        pltpu.VMEM((_CHUNK, _H), _f32),
        pltpu.VMEM_SHARED((_NP, _H), _f32),
        pltpu.SemaphoreType.DMA,
        pltpu.SemaphoreType.DMA,
    ],
)
def _sc_scatter(y_hbm, ei_hbm, zh_hbm, agg_out, sidx, didx, rows0, rows1, acc,
                sem0, sem1):
    c = lax.axis_index("c")
    s = lax.axis_index("s")
    rbase = pl.multiple_of(s * _RPT, 8)
    pltpu.sync_copy(zh_hbm.at[pl.ds(rbase, _RPT)], acc.at[pl.ds(rbase, _RPT)])
    tid = s * _NC + c
    ebase = pl.multiple_of(tid * _NCHUNK, 8)
    pltpu.sync_copy(ei_hbm.at[0, pl.ds(ebase, 2 * _QC)], sidx.at[pl.ds(0, 2 * _QC)])
    pltpu.sync_copy(ei_hbm.at[1, pl.ds(ebase, 2 * _QC)], didx.at[pl.ds(0, 2 * _QC)])
    plsc.subcore_barrier()

    pltpu.async_copy(y_hbm.at[sidx.at[0]], rows0, sem0)

    def body(k, carry):
        g0 = 2 * k
        g1 = 2 * k + 1

        @pl.when(lax.rem(g0, _QC) == 0)
        def _():
            q = g0 // _QC

            @pl.when(q + 2 < _NQ)
            def _():
                src = pl.multiple_of(ebase + (q + 2) * _QC, 8)
                dst = pl.multiple_of(lax.rem(q + 2, 3) * _QC, 8)
                pltpu.sync_copy(ei_hbm.at[0, pl.ds(src, _QC)],
                                sidx.at[pl.ds(dst, _QC)])
                pltpu.sync_copy(ei_hbm.at[1, pl.ds(src, _QC)],
                                didx.at[pl.ds(dst, _QC)])

        r0 = lax.rem(g0, _RS)
        r1 = lax.rem(g1, _RS)
        pltpu.async_copy(y_hbm.at[sidx.at[r1]], rows1, sem1)
        pltpu.make_async_copy(y_hbm.at[sidx.at[r0]], rows0, sem0).wait()
        pltpu.sync_copy(rows0, acc.at[didx.at[r0]], add=True)

        @pl.when(k < _NCHUNK // 2 - 1)
        def _():
            pltpu.async_copy(y_hbm.at[sidx.at[lax.rem(g0 + 2, _RS)]], rows0, sem0)

        pltpu.make_async_copy(y_hbm.at[sidx.at[r1]], rows1, sem1).wait()
        pltpu.sync_copy(rows1, acc.at[didx.at[r1]], add=True)
        return carry

    lax.fori_loop(0, _NCHUNK // 2, body, 0)
    plsc.subcore_barrier()
    pltpu.sync_copy(acc.at[pl.ds(rbase, _RPT)],
                    agg_out.at[c, pl.ds(rbase, _RPT)])


def _tc_pre(x, Wl, Wr, bl):
    def body(x_ref, wl_ref, wr_ref, bl_ref, y_ref, z_ref):
        h = x_ref[...]
        y_ref[...] = jnp.dot(h, wl_ref[...], preferred_element_type=_f32)
        z_ref[...] = jnp.dot(h, wr_ref[...], preferred_element_type=_f32) + bl_ref[...]

    return pl.pallas_call(
        body,
        grid=(_GRID,),
        in_specs=[
            pl.BlockSpec((_BN, _H), lambda i: (i, 0)),
            pl.BlockSpec((_H, _H), lambda i: (0, 0)),
            pl.BlockSpec((_H, _H), lambda i: (0, 0)),
            pl.BlockSpec((1, _H), lambda i: (0, 0)),
        ],
        out_specs=[pl.BlockSpec((_BN, _H), lambda i: (i, 0))] * 2,
        out_shape=[jax.ShapeDtypeStruct((_N, _H), _f32)] * 2,
    )(x, Wl, Wr, bl.reshape(1, _H))


def _combine(aggp_ref, cntp_ref, z_ref):
    agg = aggp_ref[0] + aggp_ref[1]
    cnt = cntp_ref[0] + cntp_ref[1]
    inv = 1.0 / jnp.maximum(cnt, 1.0)
    out = agg * inv + z_ref[...]
    nrm = jnp.sqrt(jnp.sum(out * out, axis=1, keepdims=True))
    return out / jnp.maximum(nrm, 1e-12)


def _tc_post(aggp, cntp, z, Wl, Wr, bl):
    def body(aggp_ref, cntp_ref, z_ref, wl_ref, wr_ref, bl_ref, y_ref, z2_ref):
        h = jnp.maximum(_combine(aggp_ref, cntp_ref, z_ref), 0.0)
        y_ref[...] = jnp.dot(h, wl_ref[...], preferred_element_type=_f32)
        z2_ref[...] = jnp.dot(h, wr_ref[...], preferred_element_type=_f32) + bl_ref[...]

    return pl.pallas_call(
        body,
        grid=(_GRID,),
        in_specs=[
            pl.BlockSpec((_NC, _BN, _H), lambda i: (0, i, 0)),
            pl.BlockSpec((_NC, _BN, _H), lambda i: (0, i, 0)),
            pl.BlockSpec((_BN, _H), lambda i: (i, 0)),
            pl.BlockSpec((_H, _H), lambda i: (0, 0)),
            pl.BlockSpec((_H, _H), lambda i: (0, 0)),
            pl.BlockSpec((1, _H), lambda i: (0, 0)),
        ],
        out_specs=[pl.BlockSpec((_BN, _H), lambda i: (i, 0))] * 2,
        out_shape=[jax.ShapeDtypeStruct((_N, _H), _f32)] * 2,
    )(aggp, cntp, z, Wl, Wr, bl.reshape(1, _H))


def _tc_final(aggp, cntp, z):
    def body(aggp_ref, cntp_ref, z_ref, h_ref):
        h_ref[...] = _combine(aggp_ref, cntp_ref, z_ref)

    return pl.pallas_call(
        body,
        grid=(_GRID,),
        in_specs=[
            pl.BlockSpec((_NC, _BN, _H), lambda i: (0, i, 0)),
            pl.BlockSpec((_NC, _BN, _H), lambda i: (0, i, 0)),
            pl.BlockSpec((_BN, _H), lambda i: (i, 0)),
        ],
        out_specs=pl.BlockSpec((_BN, _H), lambda i: (i, 0)),
        out_shape=jax.ShapeDtypeStruct((_N, _H), _f32),
    )(aggp, cntp, z)


def kernel(x, edge_index, Wl0, bl0, Wr0, Wl1, bl1, Wr1, Wl2, bl2, Wr2):
    ei3 = edge_index.reshape(2, _E // _CHUNK, _CHUNK)
    zh = jnp.zeros((_NP, _H), _f32)
    ones = jnp.ones((_CHUNK, _H), _f32)

    cntp = _sc_cnt(ei3, zh, ones)
    y0, z0 = _tc_pre(x, Wl0, Wr0, bl0)
    aggp0 = _sc_scatter(y0, ei3, zh)
    y1, z1 = _tc_post(aggp0, cntp, z0, Wl1, Wr1, bl1)
    aggp1 = _sc_scatter(y1, ei3, zh)
    y2, z2 = _tc_post(aggp1, cntp, z1, Wl2, Wr2, bl2)
    aggp2 = _sc_scatter(y2, ei3, zh)
    return _tc_final(aggp2, cntp, z2)
